# baseline (speedup 1.0000x reference)
import functools

import jax
import jax.numpy as jnp
from jax import lax
from jax.experimental import pallas as pl
from jax.experimental.pallas import tpu as pltpu
from jax.experimental.pallas import tpu_sc as plsc

N = 10000
D = 128
E = 320000

NC = 2
NS = 16
NW = NC * NS
CHUNK = 80
NBUF = 3
NGROUP = 42
NCHUNK = NBUF * NGROUP
EPW = NCHUNK * CHUNK
EPAD = NW * EPW

NP_ROWS = 10240
RPT = NP_ROWS // NS

DEG_FIRE = 21


def _pad_edges(edge_index):
    pad = EPAD - E
    it = lax.iota(jnp.int32, pad)
    prow = it % N
    pcol = N + it % (NP_ROWS - N)
    row = jnp.concatenate([edge_index[0], prow])
    col = jnp.concatenate([edge_index[1], pcol])
    return row, col


def _sc_degree(col3d):
    mesh = plsc.VectorSubcoreMesh(core_axis_name="c", subcore_axis_name="s")

    @functools.partial(
        pl.kernel,
        out_type=jax.ShapeDtypeStruct((NC, NP_ROWS), jnp.float32),
        mesh=mesh,
        scratch_types=[
            pltpu.VMEM((NCHUNK, CHUNK), jnp.int32),
            pltpu.VMEM((CHUNK,), jnp.float32),
            pltpu.VMEM((RPT,), jnp.float32),
            pltpu.VMEM_SHARED((NP_ROWS,), jnp.float32),
            pltpu.SemaphoreType.DMA,
        ],
    )
    def k(col_hbm, deg_out, idx_v, ones_v, zero_v, deg_sh, sem):
        cid = lax.axis_index("c")
        sid = lax.axis_index("s")
        wid = sid * NC + cid
        pltpu.sync_copy(col_hbm.at[wid], idx_v)

        for j in range(CHUNK // 16):
            ones_v[pl.ds(j * 16, 16)] = jnp.ones((16,), jnp.float32)

        def zfill(i, carry):
            zero_v[pl.ds(i * 16, 16)] = jnp.zeros((16,), jnp.float32)
            return carry

        lax.fori_loop(0, RPT // 16, zfill, 0)
        pltpu.sync_copy(zero_v, deg_sh.at[pl.ds(sid * RPT, RPT)])
        plsc.subcore_barrier()

        def body(grp, carry):
            for j in range(DEG_FIRE):
                c = grp * DEG_FIRE + j
                pltpu.async_copy(ones_v, deg_sh.at[idx_v.at[c]], sem,
                                 add=True)
            for j in range(DEG_FIRE):
                c = grp * DEG_FIRE + j
                pltpu.make_async_copy(ones_v, deg_sh.at[idx_v.at[c]],
                                      sem).wait()
            return carry

        lax.fori_loop(0, NCHUNK // DEG_FIRE, body, 0)
        plsc.subcore_barrier()
        pltpu.sync_copy(deg_sh.at[pl.ds(sid * RPT, RPT)],
                        deg_out.at[cid, pl.ds(sid * RPT, RPT)])

    return k(col3d)


def _tc_linear(x, wt, degp_t):
    BR = 400

    def body(x_ref, wt_ref, deg_ref, o_ref):
        degb = deg_ref[...]
        deg = degb[:, 0:1] + degb[:, 1:2]
        dinv = jnp.where(deg > 0, lax.rsqrt(deg), 0.0)
        o_ref[...] = jnp.dot(x_ref[...] * dinv, wt_ref[...],
                             preferred_element_type=jnp.float32)

    return pl.pallas_call(
        body,
        grid=(N // BR,),
        in_specs=[
            pl.BlockSpec((BR, D), lambda i: (i, 0)),
            pl.BlockSpec((D, D), lambda i: (0, 0)),
            pl.BlockSpec((BR, 2), lambda i: (i, 0)),
        ],
        out_specs=pl.BlockSpec((BR, D), lambda i: (i, 0)),
        out_shape=jax.ShapeDtypeStruct((N, D), jnp.float32),
    )(x, wt, degp_t)


def _sc_gather_scatter(h, row4d, col4d):
    mesh = plsc.VectorSubcoreMesh(core_axis_name="c", subcore_axis_name="s")

    @functools.partial(
        pl.kernel,
        out_type=jax.ShapeDtypeStruct((NC, NP_ROWS, D), jnp.float32),
        mesh=mesh,
        scratch_types=[
            [pltpu.VMEM((CHUNK, D), jnp.float32) for _ in range(NBUF)],
            [pltpu.SemaphoreType.DMA for _ in range(NBUF)],
            [pltpu.SemaphoreType.DMA for _ in range(NBUF)],
            [pltpu.VMEM((NBUF, CHUNK), jnp.int32) for _ in range(2)],
            [pltpu.VMEM((NBUF, CHUNK), jnp.int32) for _ in range(2)],
            pltpu.SemaphoreType.DMA,
            pltpu.VMEM_SHARED((NP_ROWS, D), jnp.float32),
        ],
    )
    def k(h_hbm, row_hbm, col_hbm, out_hbm, bufs, gsem, ssem2, rsb, csb,
          ssem, acc_sh):
        cid = lax.axis_index("c")
        sid = lax.axis_index("s")
        wid = sid * NC + cid
        row0 = sid * RPT

        def zfill(i, carry):
            for j in range(D // 16):
                bufs[0][i, pl.ds(j * 16, 16)] = jnp.zeros((16,), jnp.float32)
            return carry

        lax.fori_loop(0, CHUNK, zfill, 0)
        for j in range(RPT // CHUNK):
            pltpu.sync_copy(bufs[0], acc_sh.at[pl.ds(row0 + j * CHUNK, CHUNK)])
        plsc.subcore_barrier()

        def slab_load(g, r, c_, sem):
            pltpu.async_copy(row_hbm.at[wid, g], r, sem)
            pltpu.async_copy(col_hbm.at[wid, g], c_, sem)

        def slab_wait(g, r, c_, sem):
            pltpu.make_async_copy(row_hbm.at[wid, g], r, sem).wait()
            pltpu.make_async_copy(col_hbm.at[wid, g], c_, sem).wait()

        def fire_gather(rslab, b):
            pltpu.async_copy(h_hbm.at[rslab.at[b]], bufs[b], gsem[b])

        def wait_gather(rslab, b):
            pltpu.make_async_copy(h_hbm.at[rslab.at[b]], bufs[b],
                                  gsem[b]).wait()

        def fire_scatter(cslab, b):
            pltpu.async_copy(bufs[b], acc_sh.at[cslab.at[b]], ssem2[b],
                             add=True)

        def wait_scatter(cslab, b):
            pltpu.make_async_copy(bufs[b], acc_sh.at[cslab.at[b]],
                                  ssem2[b]).wait()

        slab_load(0, rsb[0], csb[0], ssem)
        slab_wait(0, rsb[0], csb[0], ssem)
        for b in range(NBUF):
            fire_gather(rsb[0], b)
        slab_load(1, rsb[1], csb[1], ssem)

        def do_group(g, pe, po, last):
            if not last:
                slab_wait(g + 1, rsb[po], csb[po], ssem)
            for b in range(NBUF):
                wait_gather(rsb[pe], b)
                fire_scatter(csb[pe], b)
            for b in range(NBUF):
                wait_scatter(csb[pe], b)
                if not last:
                    fire_gather(rsb[po], b)
            if not last:

                @pl.when(g + 2 < NGROUP)
                def _():
                    slab_load(g + 2, rsb[pe], csb[pe], ssem)

        def pair(p, carry):
            g = p * 2
            do_group(g, 0, 1, False)
            do_group(g + 1, 1, 0, False)
            return carry

        lax.fori_loop(0, NGROUP // 2 - 1, pair, 0)
        do_group(NGROUP - 2, 0, 1, False)
        do_group(NGROUP - 1, 1, 0, True)

        plsc.subcore_barrier()
        pltpu.sync_copy(acc_sh.at[pl.ds(row0, RPT)],
                        out_hbm.at[cid, pl.ds(row0, RPT)])

    return k(h, row4d, col4d)


def _tc_epilogue(accp, degp_t, b2, pa2):
    BR = 400

    def body(a_ref, deg_ref, b_ref, pa_ref, o_ref):
        s = a_ref[0] + a_ref[1]
        degb = deg_ref[...]
        deg = degb[:, 0:1] + degb[:, 1:2]
        dinv = jnp.where(deg > 0, lax.rsqrt(deg), 0.0)
        v = s * dinv + b_ref[...]
        pa = pa_ref[0, 0]
        o_ref[...] = jnp.where(v >= 0, v, pa * v)

    return pl.pallas_call(
        body,
        grid=(N // BR,),
        in_specs=[
            pl.BlockSpec((2, BR, D), lambda i: (0, i, 0)),
            pl.BlockSpec((BR, 2), lambda i: (i, 0)),
            pl.BlockSpec((1, D), lambda i: (0, 0)),
            pl.BlockSpec((1, 1), lambda i: (0, 0)),
        ],
        out_specs=pl.BlockSpec((BR, D), lambda i: (i, 0)),
        out_shape=jax.ShapeDtypeStruct((N, D), jnp.float32),
    )(accp, degp_t, b2, pa2)


def kernel(x, edge_index, W, b, prelu_a):
    row, col = _pad_edges(edge_index)
    row4d = row.reshape(NW, NGROUP, NBUF, CHUNK)
    col4d = col.reshape(NW, NGROUP, NBUF, CHUNK)

    degp = _sc_degree(col4d.reshape(NW, NCHUNK, CHUNK))
    degp_t = degp.T[:N]
    h = _tc_linear(x, W.T, degp_t)
    accp = _sc_gather_scatter(h, row4d, col4d)
    out = _tc_epilogue(accp[:, :N], degp_t,
                       b.reshape(1, D), prelu_a.reshape(1, 1))
    return out


# speedup vs baseline: 1.2232x; 1.2232x over previous
import functools

import jax
import jax.numpy as jnp
from jax import lax
from jax.experimental import pallas as pl
from jax.experimental.pallas import tpu as pltpu
from jax.experimental.pallas import tpu_sc as plsc

N = 10000
D = 128
E = 320000

NC = 2
NS = 16
NW = NC * NS
CHUNK = 40
NBUF = 5
EPW = E // NW
NCHUNK = EPW // CHUNK
NGROUP = NCHUNK // NBUF
TOTCHUNK = E // CHUNK

NP_ROWS = 10240
RPT = NP_ROWS // NS

DEG_FIRE = 25


def _sc_degree(edge5d):
    mesh = plsc.VectorSubcoreMesh(core_axis_name="c", subcore_axis_name="s")

    @functools.partial(
        pl.kernel,
        out_type=jax.ShapeDtypeStruct((NC, NP_ROWS), jnp.float32),
        mesh=mesh,
        scratch_types=[
            pltpu.VMEM((NGROUP, NBUF, CHUNK), jnp.int32),
            pltpu.VMEM((CHUNK,), jnp.float32),
            pltpu.VMEM((RPT,), jnp.float32),
            pltpu.VMEM_SHARED((NP_ROWS,), jnp.float32),
            pltpu.SemaphoreType.DMA,
        ],
    )
    def k(edge_hbm, deg_out, idx_v, ones_v, zero_v, deg_sh, sem):
        cid = lax.axis_index("c")
        sid = lax.axis_index("s")
        wid = sid * NC + cid
        pltpu.sync_copy(edge_hbm.at[1, wid], idx_v)

        for j in range(CHUNK // 16):
            ones_v[pl.ds(j * 16, 16)] = jnp.ones((16,), jnp.float32)

        def zfill(i, carry):
            zero_v[pl.ds(i * 16, 16)] = jnp.zeros((16,), jnp.float32)
            return carry

        lax.fori_loop(0, RPT // 16, zfill, 0)
        pltpu.sync_copy(zero_v, deg_sh.at[pl.ds(sid * RPT, RPT)])
        plsc.subcore_barrier()

        def body(sg, carry):
            for j in range(DEG_FIRE // NBUF):
                for b in range(NBUF):
                    g = sg * (DEG_FIRE // NBUF) + j
                    pltpu.async_copy(ones_v, deg_sh.at[idx_v.at[g, b]], sem,
                                     add=True)
            for j in range(DEG_FIRE // NBUF):
                for b in range(NBUF):
                    g = sg * (DEG_FIRE // NBUF) + j
                    pltpu.make_async_copy(ones_v, deg_sh.at[idx_v.at[g, b]],
                                          sem).wait()
            return carry

        lax.fori_loop(0, NGROUP // (DEG_FIRE // NBUF), body, 0)
        plsc.subcore_barrier()
        pltpu.sync_copy(deg_sh.at[pl.ds(sid * RPT, RPT)],
                        deg_out.at[cid, pl.ds(sid * RPT, RPT)])

    return k(edge5d)


def _tc_linear(x, wt, degp):
    BR = 2000

    def body(x_ref, wt_ref, deg_ref, o_ref):
        degb = deg_ref[...]
        deg = degb[:, 0:1] + degb[:, 1:2]
        dinv = jnp.where(deg > 0, lax.rsqrt(deg), 0.0)
        o_ref[...] = jnp.dot(x_ref[...] * dinv, wt_ref[...],
                             preferred_element_type=jnp.float32)

    return pl.pallas_call(
        body,
        grid=(N // BR,),
        in_specs=[
            pl.BlockSpec((BR, D), lambda i: (i, 0)),
            pl.BlockSpec((D, D), lambda i: (0, 0)),
            pl.BlockSpec((BR, 2), lambda i: (i, 0)),
        ],
        out_specs=pl.BlockSpec((BR, D), lambda i: (i, 0)),
        out_shape=jax.ShapeDtypeStruct((N, D), jnp.float32),
    )(x, wt, degp)


def _sc_gather_scatter(h, edge5d):
    mesh = plsc.VectorSubcoreMesh(core_axis_name="c", subcore_axis_name="s")

    @functools.partial(
        pl.kernel,
        out_type=jax.ShapeDtypeStruct((NC, NP_ROWS, D), jnp.float32),
        mesh=mesh,
        scratch_types=[
            [pltpu.VMEM((CHUNK, D), jnp.float32) for _ in range(NBUF)],
            [pltpu.SemaphoreType.DMA for _ in range(NBUF)],
            [pltpu.VMEM((NBUF, CHUNK), jnp.int32) for _ in range(2)],
            [pltpu.VMEM((NBUF, CHUNK), jnp.int32) for _ in range(2)],
            pltpu.SemaphoreType.DMA,
            pltpu.VMEM_SHARED((NP_ROWS, D), jnp.float32),
        ],
    )
    def k(h_hbm, edge_hbm, out_hbm, bufs, gsem, rsb, csb, ssem, acc_sh):
        cid = lax.axis_index("c")
        sid = lax.axis_index("s")
        wid = sid * NC + cid
        row0 = sid * RPT

        def zfill(i, carry):
            for j in range(D // 16):
                bufs[0][i, pl.ds(j * 16, 16)] = jnp.zeros((16,), jnp.float32)
            return carry

        lax.fori_loop(0, CHUNK, zfill, 0)
        for j in range(RPT // CHUNK):
            pltpu.sync_copy(bufs[0], acc_sh.at[pl.ds(row0 + j * CHUNK, CHUNK)])
        plsc.subcore_barrier()

        def slab_load(g, r, c_, sem):
            pltpu.async_copy(edge_hbm.at[0, wid, g], r, sem)
            pltpu.async_copy(edge_hbm.at[1, wid, g], c_, sem)

        def slab_wait(g, r, c_, sem):
            pltpu.make_async_copy(edge_hbm.at[0, wid, g], r, sem).wait()
            pltpu.make_async_copy(edge_hbm.at[1, wid, g], c_, sem).wait()

        def fire_gather(rslab, b):
            pltpu.async_copy(h_hbm.at[rslab.at[b]], bufs[b], gsem[b])

        def wait_gather(rslab, b):
            pltpu.make_async_copy(h_hbm.at[rslab.at[b]], bufs[b],
                                  gsem[b]).wait()

        def scatter(cslab, b):
            pltpu.sync_copy(bufs[b], acc_sh.at[cslab.at[b]], add=True)

        slab_load(0, rsb[0], csb[0], ssem)
        slab_wait(0, rsb[0], csb[0], ssem)
        for b in range(NBUF):
            fire_gather(rsb[0], b)
        slab_load(1, rsb[1], csb[1], ssem)

        def do_group(g, pe, po, last):
            if not last:
                slab_wait(g + 1, rsb[po], csb[po], ssem)
            for b in range(NBUF):
                wait_gather(rsb[pe], b)
                scatter(csb[pe], b)
                if not last:
                    fire_gather(rsb[po], b)
            if not last:

                @pl.when(g + 2 < NGROUP)
                def _():
                    slab_load(g + 2, rsb[pe], csb[pe], ssem)

        def pair(p, carry):
            g = p * 2
            do_group(g, 0, 1, False)
            do_group(g + 1, 1, 0, False)
            return carry

        lax.fori_loop(0, NGROUP // 2 - 1, pair, 0)
        do_group(NGROUP - 2, 0, 1, False)
        do_group(NGROUP - 1, 1, 0, True)

        plsc.subcore_barrier()
        pltpu.sync_copy(acc_sh.at[pl.ds(row0, RPT)],
                        out_hbm.at[cid, pl.ds(row0, RPT)])

    return k(h, edge5d)


def _tc_epilogue(accp, degp, b2, pa2):
    BR = 2000

    def body(a_ref, deg_ref, b_ref, pa_ref, o_ref):
        s = a_ref[0] + a_ref[1]
        degb = deg_ref[...]
        deg = degb[:, 0:1] + degb[:, 1:2]
        dinv = jnp.where(deg > 0, lax.rsqrt(deg), 0.0)
        v = s * dinv + b_ref[...]
        pa = pa_ref[0, 0]
        o_ref[...] = jnp.where(v >= 0, v, pa * v)

    return pl.pallas_call(
        body,
        grid=(N // BR,),
        in_specs=[
            pl.BlockSpec((2, BR, D), lambda i: (0, i, 0)),
            pl.BlockSpec((BR, 2), lambda i: (i, 0)),
            pl.BlockSpec((1, D), lambda i: (0, 0)),
            pl.BlockSpec((1, 1), lambda i: (0, 0)),
        ],
        out_specs=pl.BlockSpec((BR, D), lambda i: (i, 0)),
        out_shape=jax.ShapeDtypeStruct((N, D), jnp.float32),
    )(accp, degp, b2, pa2)


def kernel(x, edge_index, W, b, prelu_a):
    edge5d = edge_index.reshape(2, NW, NGROUP, NBUF, CHUNK)
    degp = _sc_degree(edge5d)
    degp_t = degp.T
    h = _tc_linear(x, W.T, degp_t)
    accp = _sc_gather_scatter(h, edge5d)
    out = _tc_epilogue(accp, degp_t,
                       b.reshape(1, D), prelu_a.reshape(1, 1))
    return out
